# baseline (speedup 1.0000x reference)
import jax
import jax.numpy as jnp
from jax.experimental import pallas as pl
import jax.experimental.pallas.tpu as pltpu

N = 4096
B = 1024
SEQ_DIM = 1024
NODE_FEATS = 64
HIDDEN_DIM = 64

BM = 512
NB = N // BM


def _fused_kernel(nodes_ref, adj_ref, emb_ref, labels_ref,
                  w_seq_ref, b_seq_ref, w_emb_ref, b_emb_ref,
                  w_gc_ref, b_gc_ref,
                  logits_ref, loss_ref,
                  x_scr, gout_scr, seq_scr):
    i = pl.program_id(0)

    @pl.when(i < NB)
    def _phase_b():
        nf = jnp.dot(nodes_ref[...], w_emb_ref[...],
                     preferred_element_type=jnp.float32) + b_emb_ref[...]
        norm = jnp.sqrt(jnp.sum(nf * nf, axis=1, keepdims=True))
        nf = nf / jnp.maximum(norm, 1e-12)
        x_scr[pl.ds(i * BM, BM), :] = jnp.dot(
            nf, w_gc_ref[...], preferred_element_type=jnp.float32)

    @pl.when(i == NB)
    def _seq_mlp():
        seq_scr[...] = jnp.dot(emb_ref[...], w_seq_ref[...],
                               preferred_element_type=jnp.float32) + b_seq_ref[...]
        loss_ref[...] = jnp.zeros_like(loss_ref)

    k = i - NB

    @pl.when((i >= NB) & (k % 2 == 0))
    def _phase_c():
        j = k // 2
        acc = jnp.dot(adj_ref[...], x_scr[...],
                      preferred_element_type=jnp.float32) + b_gc_ref[...]
        gout_scr[pl.ds(j * BM, BM), :] = jnp.maximum(acc, 0.0)

    @pl.when((i >= NB) & (k % 2 == 1))
    def _phase_d():
        j = k // 2
        g = gout_scr[pl.ds(j * BM, BM), :]
        z = jax.lax.dot_general(
            seq_scr[...], g,
            dimension_numbers=(((1,), (1,)), ((), ())),
            preferred_element_type=jnp.float32)
        logits_ref[...] = z
        y = labels_ref[...]
        part = jnp.maximum(z, 0.0) - z * y + jnp.log1p(jnp.exp(-jnp.abs(z)))
        loss_ref[...] += jnp.sum(part).reshape(1, 1)


@jax.jit
def kernel(embeddings, labels, nodesMat, adjMat, W_seq, b_seq, W_emb, b_emb,
           W_gc, b_gc):
    b_seq2 = b_seq.reshape(1, HIDDEN_DIM)
    b_emb2 = b_emb.reshape(1, NODE_FEATS)
    b_gc2 = b_gc.reshape(1, HIDDEN_DIM)

    def clamp(v, lo, hi):
        return jnp.minimum(jnp.maximum(v, lo), hi)

    logits, loss_sum = pl.pallas_call(
        _fused_kernel,
        grid=(3 * NB,),
        in_specs=[
            pl.BlockSpec((BM, N), lambda i: (clamp(i, 0, NB - 1), 0)),
            pl.BlockSpec((BM, N), lambda i: (clamp((i - NB) // 2, 0, NB - 1), 0)),
            pl.BlockSpec((B, SEQ_DIM), lambda i: (0, 0)),
            pl.BlockSpec((B, BM), lambda i: (0, clamp((i - NB - 1) // 2, 0, NB - 1))),
            pl.BlockSpec((SEQ_DIM, HIDDEN_DIM), lambda i: (0, 0)),
            pl.BlockSpec((1, HIDDEN_DIM), lambda i: (0, 0)),
            pl.BlockSpec((N, NODE_FEATS), lambda i: (0, 0)),
            pl.BlockSpec((1, NODE_FEATS), lambda i: (0, 0)),
            pl.BlockSpec((NODE_FEATS, HIDDEN_DIM), lambda i: (0, 0)),
            pl.BlockSpec((1, HIDDEN_DIM), lambda i: (0, 0)),
        ],
        out_specs=[
            pl.BlockSpec((B, BM), lambda i: (0, clamp((i - NB - 1) // 2, 0, NB - 1))),
            pl.BlockSpec((1, 1), lambda i: (0, 0)),
        ],
        out_shape=[
            jax.ShapeDtypeStruct((B, N), jnp.float32),
            jax.ShapeDtypeStruct((1, 1), jnp.float32),
        ],
        scratch_shapes=[
            pltpu.VMEM((N, HIDDEN_DIM), jnp.float32),
            pltpu.VMEM((N, HIDDEN_DIM), jnp.float32),
            pltpu.VMEM((B, HIDDEN_DIM), jnp.float32),
        ],
    )(nodesMat, adjMat, embeddings, labels,
      W_seq, b_seq2, W_emb, b_emb2, W_gc, b_gc2)

    loss = loss_sum[0, 0] / (B * N)
    return (loss, logits)


# speedup vs baseline: 1.1279x; 1.1279x over previous
import jax
import jax.numpy as jnp
from jax.experimental import pallas as pl
import jax.experimental.pallas.tpu as pltpu

N = 4096
B = 1024
SEQ_DIM = 1024
NODE_FEATS = 64
HIDDEN_DIM = 64

BM = 512
NB = N // BM


def _fused_kernel(nodes_ref, adj_ref, emb_ref, labels_ref,
                  w_seq_ref, b_seq_ref, w_emb_ref, b_emb_ref,
                  w_gc_ref, b_gc_ref,
                  logits_ref, loss_ref,
                  x_scr, gout_scr, seq_scr):
    i = pl.program_id(0)

    @pl.when(i < NB)
    def _phase_b():
        nf = jnp.dot(nodes_ref[...], w_emb_ref[...],
                     preferred_element_type=jnp.float32) + b_emb_ref[...]
        norm = jnp.sqrt(jnp.sum(nf * nf, axis=1, keepdims=True))
        nf = nf / jnp.maximum(norm, 1e-12)
        x_scr[pl.ds(i * BM, BM), :] = jnp.dot(
            nf, w_gc_ref[...], preferred_element_type=jnp.float32)

    @pl.when(i == NB)
    def _seq_mlp():
        seq_scr[...] = jnp.dot(emb_ref[...], w_seq_ref[...],
                               preferred_element_type=jnp.float32) + b_seq_ref[...]
        loss_ref[...] = jnp.zeros_like(loss_ref)

    @pl.when((i >= NB) & (i < 2 * NB))
    def _phase_c():
        j = i - NB
        acc = jnp.dot(adj_ref[...], x_scr[...],
                      preferred_element_type=jnp.float32) + b_gc_ref[...]
        gout_scr[pl.ds(j * BM, BM), :] = jnp.maximum(acc, 0.0)

    @pl.when(i > NB)
    def _phase_d():
        j = i - NB - 1
        g = gout_scr[pl.ds(j * BM, BM), :]
        z = jax.lax.dot_general(
            seq_scr[...], g,
            dimension_numbers=(((1,), (1,)), ((), ())),
            preferred_element_type=jnp.float32)
        logits_ref[...] = z
        y = labels_ref[...]
        part = jnp.maximum(z, 0.0) - z * y + jnp.log1p(jnp.exp(-jnp.abs(z)))
        loss_ref[...] += jnp.sum(part).reshape(1, 1)


@jax.jit
def kernel(embeddings, labels, nodesMat, adjMat, W_seq, b_seq, W_emb, b_emb,
           W_gc, b_gc):
    b_seq2 = b_seq.reshape(1, HIDDEN_DIM)
    b_emb2 = b_emb.reshape(1, NODE_FEATS)
    b_gc2 = b_gc.reshape(1, HIDDEN_DIM)

    def clamp(v, lo, hi):
        return jnp.minimum(jnp.maximum(v, lo), hi)

    logits, loss_sum = pl.pallas_call(
        _fused_kernel,
        grid=(2 * NB + 1,),
        in_specs=[
            pl.BlockSpec((BM, N), lambda i: (clamp(i, 0, NB - 1), 0)),
            pl.BlockSpec((BM, N), lambda i: (clamp(i - NB, 0, NB - 1), 0)),
            pl.BlockSpec((B, SEQ_DIM), lambda i: (0, 0)),
            pl.BlockSpec((B, BM), lambda i: (0, clamp(i - NB - 1, 0, NB - 1))),
            pl.BlockSpec((SEQ_DIM, HIDDEN_DIM), lambda i: (0, 0)),
            pl.BlockSpec((1, HIDDEN_DIM), lambda i: (0, 0)),
            pl.BlockSpec((N, NODE_FEATS), lambda i: (0, 0)),
            pl.BlockSpec((1, NODE_FEATS), lambda i: (0, 0)),
            pl.BlockSpec((NODE_FEATS, HIDDEN_DIM), lambda i: (0, 0)),
            pl.BlockSpec((1, HIDDEN_DIM), lambda i: (0, 0)),
        ],
        out_specs=[
            pl.BlockSpec((B, BM), lambda i: (0, clamp(i - NB - 1, 0, NB - 1))),
            pl.BlockSpec((1, 1), lambda i: (0, 0)),
        ],
        out_shape=[
            jax.ShapeDtypeStruct((B, N), jnp.float32),
            jax.ShapeDtypeStruct((1, 1), jnp.float32),
        ],
        scratch_shapes=[
            pltpu.VMEM((N, HIDDEN_DIM), jnp.float32),
            pltpu.VMEM((N, HIDDEN_DIM), jnp.float32),
            pltpu.VMEM((B, HIDDEN_DIM), jnp.float32),
        ],
    )(nodesMat, adjMat, embeddings, labels,
      W_seq, b_seq2, W_emb, b_emb2, W_gc, b_gc2)

    loss = loss_sum[0, 0] / (B * N)
    return (loss, logits)
